# Initial kernel scaffold; baseline (speedup 1.0000x reference)
import jax
import jax.numpy as jnp
from jax.experimental import pallas as pl


def kernel(x, gate_W, gate_b, expert_W, expert_b):
    raise NotImplementedError("write your pallas kernel here")


# speedup vs baseline: 1.2339x; 1.2339x over previous
import jax
import jax.numpy as jnp
from jax.experimental import pallas as pl
from jax.experimental.pallas import tpu as pltpu

E = 8
K = 2
D_IN = 1024
D_OUT = 1024
TM = 512


def _moe_body(x_ref, gw_ref, gb_ref, ew_ref, eb_ref, out_ref, w_scr):
    e = pl.program_id(1)

    @pl.when(e == 0)
    def _gate():
        xv = x_ref[...]
        logits = jax.lax.dot_general(
            xv, gw_ref[...], (((1,), (1,)), ((), ())),
            preferred_element_type=jnp.float32) + gb_ref[...]
        iota = jax.lax.broadcasted_iota(jnp.int32, logits.shape, 1)
        m1 = jnp.max(logits, axis=1, keepdims=True)
        i1 = jnp.min(jnp.where(logits == m1, iota, E), axis=1, keepdims=True)
        masked = jnp.where(iota == i1, -jnp.inf, logits)
        m2 = jnp.max(masked, axis=1, keepdims=True)
        i2 = jnp.min(jnp.where(masked == m2, iota, E), axis=1, keepdims=True)
        t = jnp.exp(m2 - m1)
        w1 = 1.0 / (1.0 + t)
        w2 = t / (1.0 + t)
        w_scr[...] = jnp.where(iota == i1, w1, 0.0) + jnp.where(iota == i2, w2, 0.0)
        out_ref[...] = jnp.zeros_like(out_ref)

    y = jax.lax.dot_general(
        x_ref[...], ew_ref[0], (((1,), (1,)), ((), ())),
        preferred_element_type=jnp.float32) + eb_ref[0]
    lane = jax.lax.broadcasted_iota(jnp.int32, (TM, E), 1)
    w_e = jnp.sum(jnp.where(lane == e, w_scr[...], 0.0), axis=1, keepdims=True)
    out_ref[...] += w_e * y


def kernel(x, gate_W, gate_b, expert_W, expert_b):
    T = x.shape[0] * x.shape[1]
    x_flat = x.reshape(T, D_IN)
    out = pl.pallas_call(
        _moe_body,
        grid=(T // TM, E),
        in_specs=[
            pl.BlockSpec((TM, D_IN), lambda m, e: (m, 0)),
            pl.BlockSpec((E, D_IN), lambda m, e: (0, 0)),
            pl.BlockSpec((1, E), lambda m, e: (0, 0)),
            pl.BlockSpec((1, D_OUT, D_IN), lambda m, e: (e, 0, 0)),
            pl.BlockSpec((1, 1, D_OUT), lambda m, e: (e, 0, 0)),
        ],
        out_specs=pl.BlockSpec((TM, D_OUT), lambda m, e: (m, 0)),
        out_shape=jax.ShapeDtypeStruct((T, D_OUT), jnp.float32),
        scratch_shapes=[pltpu.VMEM((TM, E), jnp.float32)],
        compiler_params=pltpu.CompilerParams(
            dimension_semantics=("arbitrary", "arbitrary")),
    )(x_flat, gate_W, gate_b.reshape(1, E), expert_W,
      expert_b.reshape(E, 1, D_OUT))
    return out.reshape(*x.shape[:-1], D_OUT)
